# Initial kernel scaffold; baseline (speedup 1.0000x reference)
import jax
import jax.numpy as jnp
from jax.experimental import pallas as pl


def kernel(x, tokens, params):
    raise NotImplementedError("write your pallas kernel here")


# speedup vs baseline: 6.7985x; 6.7985x over previous
import functools

import jax
import jax.numpy as jnp
from jax.experimental import pallas as pl
from jax.experimental.pallas import tpu as pltpu

_RADII = (0.05, 0.1)
_KS = (8, 16)
_HEADS = 4
_DIM_HEAD = 64
_SLICE = 32
_HID = 256
_KMAX = 16
_BIG = 3.0e38


def _ball_mlp_body(tok_ref, geoT_ref, iot_ref, wd_ref, wx_ref, wy_ref, wz_ref,
                   b1_ref, w2_ref, b2_ref, w3_ref, b3_ref, mw_ref, mb_ref,
                   out_ref, d2_ref):
    TT = tok_ref.shape[0]
    tx = tok_ref[:, 0:1]
    ty = tok_ref[:, 1:2]
    tz = tok_ref[:, 2:3]
    ddx = tx - geoT_ref[0:1, :]
    ddy = ty - geoT_ref[1:2, :]
    ddz = tz - geoT_ref[2:3, :]
    d2_ref[...] = (ddx * ddx + ddy * ddy) + ddz * ddz

    slot = jax.lax.broadcasted_iota(jnp.int32, (1, _KMAX), 1)
    zeros16 = jnp.zeros((TT, _KMAX), jnp.float32)

    def pass_body(p, carry):
        dv, gxv, gyv, gzv = carry
        d2c = d2_ref[...]
        iot = iot_ref[...]
        m = jnp.min(d2c, axis=-1, keepdims=True)
        eqm = d2c == m
        idxm = jnp.min(jnp.where(eqm, iot, _BIG), axis=-1, keepdims=True)
        kill = eqm & (iot == idxm)
        d2_ref[...] = jnp.where(kill, _BIG, d2c)
        oh = (slot == p).astype(jnp.float32)
        gxm = jnp.max(jnp.where(kill, geoT_ref[0:1, :], -1.0),
                      axis=-1, keepdims=True)
        gym = jnp.max(jnp.where(kill, geoT_ref[1:2, :], -1.0),
                      axis=-1, keepdims=True)
        gzm = jnp.max(jnp.where(kill, geoT_ref[2:3, :], -1.0),
                      axis=-1, keepdims=True)
        return (dv + m * oh, gxv + gxm * oh, gyv + gym * oh, gzv + gzm * oh)

    d2s, gxs, gys, gzs = jax.lax.fori_loop(
        0, _KMAX, pass_body, (zeros16, zeros16, zeros16, zeros16))

    d = jnp.sqrt(jnp.maximum(d2s, 1e-12))
    dxs = gxs - tx
    dys = gys - ty
    dzs = gzs - tz

    lane = jax.lax.broadcasted_iota(jnp.int32, d.shape, 1)
    dims = (((1,), (0,)), ((), ()))
    outs = []
    for j in range(2):
        valid = d <= _RADII[j]
        if _KS[j] < _KMAX:
            valid = valid & (lane < _KS[j])
        vf = valid.astype(jnp.float32)
        z = (jax.lax.dot_general(d * vf, wd_ref[j], dims,
                                 preferred_element_type=jnp.float32)
             + jax.lax.dot_general(dxs * vf, wx_ref[j], dims,
                                   preferred_element_type=jnp.float32)
             + jax.lax.dot_general(dys * vf, wy_ref[j], dims,
                                   preferred_element_type=jnp.float32)
             + jax.lax.dot_general(dzs * vf, wz_ref[j], dims,
                                   preferred_element_type=jnp.float32)
             + b1_ref[j])
        z = jax.nn.gelu(z)
        z = jax.lax.dot_general(z, w2_ref[j], dims,
                                preferred_element_type=jnp.float32) + b2_ref[j]
        z = jax.nn.gelu(z)
        z = jax.lax.dot_general(z, w3_ref[j], dims,
                                preferred_element_type=jnp.float32) + b3_ref[j]
        outs.append(z)

    h = jnp.concatenate(outs, axis=1)
    out_ref[...] = (jax.lax.dot_general(h, mw_ref[...], dims,
                                        preferred_element_type=jnp.float32)
                    + mb_ref[...])


def _layernorm(x, g, b):
    m = jnp.mean(x, axis=-1, keepdims=True)
    v = jnp.mean((x - m) * (x - m), axis=-1, keepdims=True)
    return (x - m) / jnp.sqrt(v + 1e-5) * g + b


def _softmax(z):
    z = z - jnp.max(z, axis=-1, keepdims=True)
    e = jnp.exp(z)
    return e / jnp.sum(e, axis=-1, keepdims=True)


def _transolver_body(h_ref, ln1g_ref, ln1b_ref, fxw_ref, fxb_ref, xw_ref,
                     xb_ref, slw_ref, slb_ref, temp_ref, qw_ref, kw_ref,
                     vw_ref, ow_ref, ob_ref, ln2g_ref, ln2b_ref, mw1_ref,
                     mb1_ref, mw2_ref, mb2_ref, ln3g_ref, ln3b_ref,
                     pw_ref, pb_ref, out_ref):
    N = h_ref.shape[0]
    dims = (((1,), (0,)), ((), ()))
    dimsT = (((0,), (0,)), ((), ()))
    dimsR = (((1,), (1,)), ((), ()))
    ones_n = jnp.ones((N, 1), jnp.float32)
    scale = _DIM_HEAD ** -0.5

    x = h_ref[...]
    for j in range(2):
        xn = _layernorm(x, ln1g_ref[j], ln1b_ref[j])
        acc = ob_ref[j]
        for hh in range(_HEADS):
            cs = hh * _DIM_HEAD
            fxh = (jax.lax.dot_general(xn, fxw_ref[j, :, cs:cs + _DIM_HEAD],
                                       dims, preferred_element_type=jnp.float32)
                   + fxb_ref[j, :, cs:cs + _DIM_HEAD])
            xmh = (jax.lax.dot_general(xn, xw_ref[j, :, cs:cs + _DIM_HEAD],
                                       dims, preferred_element_type=jnp.float32)
                   + xb_ref[j, :, cs:cs + _DIM_HEAD])
            logit = (jax.lax.dot_general(xmh, slw_ref[j], dims,
                                         preferred_element_type=jnp.float32)
                     + slb_ref[j]) / temp_ref[j, 0, hh]
            sw = _softmax(logit)
            stok = jax.lax.dot_general(sw, fxh, dimsT,
                                       preferred_element_type=jnp.float32)
            snorm = jax.lax.dot_general(sw, ones_n, dimsT,
                                        preferred_element_type=jnp.float32)
            stok = stok / (snorm + 1e-5)
            q = jax.lax.dot_general(stok, qw_ref[j], dims,
                                    preferred_element_type=jnp.float32)
            k = jax.lax.dot_general(stok, kw_ref[j], dims,
                                    preferred_element_type=jnp.float32)
            v = jax.lax.dot_general(stok, vw_ref[j], dims,
                                    preferred_element_type=jnp.float32)
            attn = _softmax(jax.lax.dot_general(
                q, k, dimsR, preferred_element_type=jnp.float32) * scale)
            out = jax.lax.dot_general(attn, v, dims,
                                      preferred_element_type=jnp.float32)
            outx = jax.lax.dot_general(sw, out, dims,
                                       preferred_element_type=jnp.float32)
            acc = acc + jax.lax.dot_general(
                outx, ow_ref[j, cs:cs + _DIM_HEAD, :], dims,
                preferred_element_type=jnp.float32)
        x = x + acc
        hn = _layernorm(x, ln2g_ref[j], ln2b_ref[j])
        z = jax.nn.gelu(jax.lax.dot_general(hn, mw1_ref[j], dims,
                                            preferred_element_type=jnp.float32)
                        + mb1_ref[j])
        x = x + (jax.lax.dot_general(z, mw2_ref[j], dims,
                                     preferred_element_type=jnp.float32)
                 + mb2_ref[j])
    x = _layernorm(x, ln3g_ref[...], ln3b_ref[...])
    out_ref[...] = (jax.lax.dot_general(x, pw_ref[...], dims,
                                        preferred_element_type=jnp.float32)
                    + pb_ref[...])


def _prep_cross(mp, k):
    w1 = mp['W1']
    wd = w1[0::7]
    wx = w1[1::7] + w1[4::7]
    wy = w1[2::7] + w1[5::7]
    wz = w1[3::7] + w1[6::7]
    pad = _KMAX - k
    if pad:
        z = jnp.zeros((pad, _HID), jnp.float32)
        wd, wx, wy, wz = (jnp.concatenate([a, z], axis=0)
                          for a in (wd, wx, wy, wz))
    return (wd, wx, wy, wz, mp['b1'][None], mp['W2'], mp['b2'][None],
            mp['W3'], mp['b3'][None])


def kernel(x, tokens, params):
    geo = x[0]
    tok = tokens[0]
    NG = geo.shape[0]
    NT = tok.shape[0]
    TT = 128 if NT % 128 == 0 else NT
    geoT = geo.T

    c0 = _prep_cross(params['cross'][0], _KS[0])
    c1 = _prep_cross(params['cross'][1], _KS[1])
    stacked = [jnp.stack([a, b]) for a, b in zip(c0, c1)]
    wd, wx, wy, wz, b1, w2, b2, w3, b3 = stacked

    def full(s):
        return pl.BlockSpec(s, lambda *i: (0,) * len(s))

    grid = NT // TT
    h = pl.pallas_call(
        _ball_mlp_body,
        grid=(grid,),
        in_specs=[
            pl.BlockSpec((TT, 3), lambda i: (i, 0)),
            full((3, NG)),
            full((1, NG)),
            full((2, _KMAX, _HID)), full((2, _KMAX, _HID)),
            full((2, _KMAX, _HID)), full((2, _KMAX, _HID)),
            full((2, 1, _HID)),
            full((2, _HID, _HID // 2)), full((2, 1, _HID // 2)),
            full((2, _HID // 2, _HID // 2)), full((2, 1, _HID // 2)),
            full((_HID, _HID)), full((1, _HID)),
        ],
        out_specs=pl.BlockSpec((TT, _HID), lambda i: (i, 0)),
        out_shape=jax.ShapeDtypeStruct((NT, _HID), jnp.float32),
        scratch_shapes=[pltpu.VMEM((TT, NG), jnp.float32)],
        compiler_params=pltpu.CompilerParams(
            dimension_semantics=("arbitrary",)),
    )(tok, geoT, jnp.arange(NG, dtype=jnp.float32)[None], wd, wx, wy, wz,
      b1, w2, b2, w3, b3, params['merge_W'], params['merge_b'][None])

    b0, bl1 = params['blk0'], params['blk1']

    def st(name):
        return jnp.stack([b0[name], bl1[name]])

    def st2(name):
        return jnp.stack([b0[name][None], bl1[name][None]])

    temp = jnp.stack([b0['temperature'].reshape(1, _HEADS),
                      bl1['temperature'].reshape(1, _HEADS)])

    operands = [
        h,
        st2('ln1_g'), st2('ln1_b'), st('fx_W'), st2('fx_b'), st('x_W'),
        st2('x_b'), st('slice_W'), st2('slice_b'), temp, st('q_W'),
        st('k_W'), st('v_W'), st('out_W'), st2('out_b'), st2('ln2_g'),
        st2('ln2_b'), st('mlp_W1'), st2('mlp_b1'), st('mlp_W2'),
        st2('mlp_b2'), bl1['ln3_g'][None], bl1['ln3_b'][None],
        bl1['proj_W'], bl1['proj_b'][None],
    ]
    out = pl.pallas_call(
        _transolver_body,
        out_shape=jax.ShapeDtypeStruct((NT, _HID), jnp.float32),
    )(*operands)
    return out[None]
